# Initial kernel scaffold; baseline (speedup 1.0000x reference)
import jax
import jax.numpy as jnp
from jax.experimental import pallas as pl


def kernel(x, edge_index, batch, W1, b1, W2, b2, W3, b3, W4, b4, W5, b5, W6, b6, lw1, lb1, lw2, lb2):
    raise NotImplementedError("write your pallas kernel here")


# speedup vs baseline: 29.0217x; 29.0217x over previous
import functools

import jax
import jax.numpy as jnp
from jax import lax
from jax.experimental import pallas as pl
from jax.experimental.pallas import tpu as pltpu
from jax.experimental.pallas import tpu_sc as plsc

_N = 10000
_NPAD = 10240
_NG = 64
_NC = 2
_NS = 16
_NW = _NC * _NS
_CHUNK = 128
_NBUF = 4
_ROWS_PT = _NPAD // _NS
_BR = 1024


def _sc_params():
  return pltpu.CompilerParams(use_tc_tiling_on_sc=False)


def _sc_mesh():
  return plsc.VectorSubcoreMesh(core_axis_name="c", subcore_axis_name="s")


@functools.lru_cache(maxsize=None)
def _deg_kernel(rounds: int):
  def body(dst_hbm, ones_hbm, zeros_hbm, out_hbm, dst_v, ones_v, acc_sh, sem):
    c = lax.axis_index("c")
    s = lax.axis_index("s")
    wid = c * _NS + s
    pltpu.sync_copy(dst_hbm.at[wid], dst_v)
    pltpu.sync_copy(ones_hbm, ones_v)
    pltpu.sync_copy(zeros_hbm, acc_sh.at[pl.ds(s * _ROWS_PT, _ROWS_PT)])
    plsc.subcore_barrier()

    def scat(j, carry):
      pltpu.async_copy(ones_v, acc_sh.at[dst_v.at[j]], sem, add=True)
      return carry

    lax.fori_loop(0, rounds, scat, 0)

    def drain(j, carry):
      pltpu.make_async_copy(ones_v, acc_sh.at[dst_v.at[j]], sem).wait()
      return carry

    lax.fori_loop(0, rounds, drain, 0)
    plsc.subcore_barrier()
    pltpu.sync_copy(acc_sh.at[pl.ds(s * _ROWS_PT, _ROWS_PT)],
                    out_hbm.at[c, pl.ds(s * _ROWS_PT, _ROWS_PT)])

  return pl.kernel(
      body,
      out_type=jax.ShapeDtypeStruct((_NC, _NPAD), jnp.float32),
      mesh=_sc_mesh(),
      scratch_types=[
          pltpu.VMEM((rounds, _CHUNK), jnp.int32),
          pltpu.VMEM((_CHUNK,), jnp.float32),
          pltpu.VMEM_SHARED((_NPAD,), jnp.float32),
          pltpu.SemaphoreType.DMA,
      ],
      compiler_params=_sc_params(),
  )


@functools.lru_cache(maxsize=None)
def _agg_kernel(fh: int, rounds: int):
  ngroups = rounds // _NBUF
  assert ngroups % 2 == 0

  def body(g_hbm, src_hbm, dst_hbm, zeros_hbm, out_hbm,
           sidx, didx, buf_v, acc_sh, isem, gsem, ssem):
    c = lax.axis_index("c")
    s = lax.axis_index("s")

    pltpu.sync_copy(zeros_hbm, acc_sh.at[pl.ds(s * _ROWS_PT, _ROWS_PT)])
    pltpu.async_copy(src_hbm.at[c, s, pl.ds(0, _NBUF)], sidx.at[0],
                     isem.at[0, 0])
    pltpu.async_copy(dst_hbm.at[s, pl.ds(0, _NBUF)], didx.at[0],
                     isem.at[0, 1])
    plsc.subcore_barrier()

    def pair(hj, carry):
      for par in range(2):
        gi = 2 * hj + par
        base = gi * _NBUF

        @pl.when(gi > 0)
        def _drain_prev():
          for b in range(_NBUF):
            pltpu.make_async_copy(buf_v.at[b], acc_sh.at[didx.at[1 - par, b]],
                                  ssem.at[b]).wait()

        @pl.when(gi + 1 < ngroups)
        def _prefetch():
          nb = (gi + 1) * _NBUF
          pltpu.async_copy(src_hbm.at[c, s, pl.ds(nb, _NBUF)],
                           sidx.at[1 - par], isem.at[1 - par, 0])
          pltpu.async_copy(dst_hbm.at[s, pl.ds(nb, _NBUF)],
                           didx.at[1 - par], isem.at[1 - par, 1])

        pltpu.make_async_copy(src_hbm.at[c, s, pl.ds(base, _NBUF)],
                              sidx.at[par], isem.at[par, 0]).wait()
        pltpu.make_async_copy(dst_hbm.at[s, pl.ds(base, _NBUF)],
                              didx.at[par], isem.at[par, 1]).wait()

        for b in range(_NBUF):
          pltpu.async_copy(g_hbm.at[sidx.at[par, b]], buf_v.at[b],
                           gsem.at[b])
        for b in range(_NBUF):
          pltpu.make_async_copy(g_hbm.at[sidx.at[par, b]], buf_v.at[b],
                                gsem.at[b]).wait()
          pltpu.async_copy(buf_v.at[b], acc_sh.at[didx.at[par, b]],
                           ssem.at[b], add=True)
      return carry

    lax.fori_loop(0, ngroups // 2, pair, 0)
    for b in range(_NBUF):
      pltpu.make_async_copy(buf_v.at[b], acc_sh.at[didx.at[1, b]],
                            ssem.at[b]).wait()
    plsc.subcore_barrier()
    pltpu.sync_copy(acc_sh.at[pl.ds(s * _ROWS_PT, _ROWS_PT)],
                    out_hbm.at[c, pl.ds(s * _ROWS_PT, _ROWS_PT)])

  return pl.kernel(
      body,
      out_type=jax.ShapeDtypeStruct((_NC, _NPAD, fh), jnp.float32),
      mesh=_sc_mesh(),
      scratch_types=[
          pltpu.VMEM((2, _NBUF, _CHUNK), jnp.int32),
          pltpu.VMEM((2, _NBUF, _CHUNK), jnp.int32),
          pltpu.VMEM((_NBUF, _CHUNK, fh), jnp.float32),
          pltpu.VMEM_SHARED((_NPAD, fh), jnp.float32),
          pltpu.SemaphoreType.DMA((2, 2)),
          pltpu.SemaphoreType.DMA((_NBUF,)),
          pltpu.SemaphoreType.DMA((_NBUF,)),
      ],
      compiler_params=_sc_params(),
  )


def _prep_call(deg_parts, x_pad, w1):
  nb = _NPAD // _BR
  fo = w1.shape[1]
  fh = fo // 2

  def body(deg_ref, x_ref, w_ref, dinv_ref, g_ref):
    i = pl.program_id(0)
    d = deg_ref[0] + deg_ref[1] + 1.0
    row = lax.broadcasted_iota(jnp.int32, (_BR, 1), 0) + i * _BR
    dinv = jnp.where(row < _N, lax.rsqrt(d), 0.0)
    dinv_ref[...] = dinv
    g = dinv * jnp.dot(x_ref[...], w_ref[...],
                       preferred_element_type=jnp.float32)
    g_ref[0] = g[:, :fh]
    g_ref[1] = g[:, fh:]

  return pl.pallas_call(
      body,
      grid=(nb,),
      in_specs=[
          pl.BlockSpec((_NC, _BR, 1), lambda i: (0, i, 0)),
          pl.BlockSpec((_BR, 128), lambda i: (i, 0)),
          pl.BlockSpec((128, fo), lambda i: (0, 0)),
      ],
      out_specs=[
          pl.BlockSpec((_BR, 1), lambda i: (i, 0)),
          pl.BlockSpec((_NC, _BR, fh), lambda i: (0, i, 0)),
      ],
      out_shape=[
          jax.ShapeDtypeStruct((_NPAD, 1), jnp.float32),
          jax.ShapeDtypeStruct((_NC, _NPAD, fh), jnp.float32),
      ],
  )(deg_parts, x_pad, w1)


def _mid_call(parts, g, dinv, b2d, w):
  fh = g.shape[2]
  fo2 = w.shape[1]
  fh2 = fo2 // 2
  nb = _NPAD // _BR

  def body(p_ref, g_ref, dinv_ref, b_ref, w_ref, o_ref):
    agg = jnp.concatenate([p_ref[0] + g_ref[0], p_ref[1] + g_ref[1]], axis=1)
    h = jnp.maximum(dinv_ref[...] * agg + b_ref[...], 0.0)
    gn = dinv_ref[...] * jnp.dot(h, w_ref[...],
                                 preferred_element_type=jnp.float32)
    o_ref[0] = gn[:, :fh2]
    o_ref[1] = gn[:, fh2:]

  return pl.pallas_call(
      body,
      grid=(nb,),
      in_specs=[
          pl.BlockSpec((_NC, _BR, fh), lambda i: (0, i, 0)),
          pl.BlockSpec((_NC, _BR, fh), lambda i: (0, i, 0)),
          pl.BlockSpec((_BR, 1), lambda i: (i, 0)),
          pl.BlockSpec((1, 2 * fh), lambda i: (0, 0)),
          pl.BlockSpec((2 * fh, fo2), lambda i: (0, 0)),
      ],
      out_specs=pl.BlockSpec((_NC, _BR, fh2), lambda i: (0, i, 0)),
      out_shape=jax.ShapeDtypeStruct((_NC, _NPAD, fh2), jnp.float32),
  )(parts, g, dinv, b2d, w)


def _final_call(parts, g, dinv, b2d, bat2d, lw1, lb1, lw2, lb2):
  nb = _NPAD // _BR
  fh = g.shape[2]

  def body(p_ref, g_ref, dinv_ref, b_ref, bat_ref, lw1_ref, lb1_ref, lw2_ref,
           lb2_ref, o_ref, acc_ref):
    i = pl.program_id(0)

    @pl.when(i == 0)
    def _init():
      acc_ref[...] = jnp.full((_NG, 128), -jnp.inf, jnp.float32)

    agg = jnp.concatenate([p_ref[0] + g_ref[0], p_ref[1] + g_ref[1]], axis=1)
    h = jnp.maximum(dinv_ref[...] * agg + b_ref[...], 0.0)
    row = lax.broadcasted_iota(jnp.int32, (_BR, 1), 0) + i * _BR
    valid = row < _N
    bat = bat_ref[...]
    bmin = jnp.min(jnp.where(valid, bat, _NG - 1))
    bmax = jnp.max(jnp.where(valid, bat, 0))
    gcol = lax.broadcasted_iota(jnp.int32, (_NG, 1), 0)

    def gbody(gg, carry):
      m = (bat == gg) & valid
      red = jnp.max(jnp.where(m, h, -jnp.inf), axis=0, keepdims=True)
      acc_ref[...] = jnp.maximum(acc_ref[...],
                                 jnp.where(gcol == gg, red, -jnp.inf))
      return carry

    lax.fori_loop(bmin, bmax + 1, gbody, 0)

    @pl.when(i == nb - 1)
    def _fin():
      z = jnp.maximum(
          jnp.dot(acc_ref[...], lw1_ref[...],
                  preferred_element_type=jnp.float32) + lb1_ref[...], 0.0)
      o_ref[...] = jnp.dot(z, lw2_ref[...],
                           preferred_element_type=jnp.float32) + lb2_ref[...]

  return pl.pallas_call(
      body,
      grid=(nb,),
      in_specs=[
          pl.BlockSpec((_NC, _BR, fh), lambda i: (0, i, 0)),
          pl.BlockSpec((_NC, _BR, fh), lambda i: (0, i, 0)),
          pl.BlockSpec((_BR, 1), lambda i: (i, 0)),
          pl.BlockSpec((1, 128), lambda i: (0, 0)),
          pl.BlockSpec((_BR, 1), lambda i: (i, 0)),
          pl.BlockSpec((128, 64), lambda i: (0, 0)),
          pl.BlockSpec((1, 64), lambda i: (0, 0)),
          pl.BlockSpec((64, 10), lambda i: (0, 0)),
          pl.BlockSpec((1, 10), lambda i: (0, 0)),
      ],
      out_specs=pl.BlockSpec((_NG, 10), lambda i: (0, 0)),
      out_shape=jax.ShapeDtypeStruct((_NG, 10), jnp.float32),
      scratch_shapes=[pltpu.VMEM((_NG, 128), jnp.float32)],
  )(parts, g, dinv, b2d, bat2d, lw1, lb1, lw2, lb2)


def _pad_idx(n):
  return _N + (jnp.arange(n, dtype=jnp.int32) % (_NPAD - _N))


def kernel(x, edge_index, batch, W1, b1, W2, b2, W3, b3, W4, b4, W5, b5,
           W6, b6, lw1, lb1, lw2, lb2):
  e = edge_index.shape[1]
  src = edge_index[0].astype(jnp.int32)
  dst = edge_index[1].astype(jnp.int32)

  r_dg = -(-e // (_NW * _CHUNK))
  pad_dg = _pad_idx(_NW * r_dg * _CHUNK - e)
  dst_dg = jnp.concatenate([dst, pad_dg]).reshape(_NW, r_dg, _CHUNK)

  r_ag = -(-e // (_NS * _CHUNK))
  r_ag = -(-r_ag // (2 * _NBUF)) * (2 * _NBUF)
  pad_ag = _pad_idx(_NS * r_ag * _CHUNK - e)
  src_t = jnp.concatenate([src, pad_ag]).reshape(_NS, r_ag, _CHUNK)
  dst_t = jnp.concatenate([dst, pad_ag]).reshape(_NS, r_ag, _CHUNK)
  srcp = jnp.stack([src_t, src_t + _NPAD])

  x_pad = jnp.pad(x, ((0, _NPAD - _N), (0, 0)))
  bat2d = jnp.pad(batch.astype(jnp.int32), (0, _NPAD - _N),
                  constant_values=_NG - 1).reshape(_NPAD, 1)
  ones_c = jnp.ones((_CHUNK,), jnp.float32)
  zeros_r = jnp.zeros((_ROWS_PT,), jnp.float32)

  deg_parts = _deg_kernel(r_dg)(dst_dg, ones_c, zeros_r)
  dinv, g = _prep_call(deg_parts.reshape(_NC, _NPAD, 1), x_pad, W1)

  ws = [W2, W3, W4, W5, W6]
  bs = [b1, b2, b3, b4, b5]
  parts = None
  for l in range(6):
    fh = g.shape[2]
    zeros_z = jnp.zeros((_ROWS_PT, fh), jnp.float32)
    parts = _agg_kernel(fh, r_ag)(g.reshape(_NC * _NPAD, fh), srcp, dst_t,
                                  zeros_z)
    if l < 5:
      g = _mid_call(parts, g, dinv, bs[l].reshape(1, -1), ws[l])

  return _final_call(parts, g, dinv, b6.reshape(1, -1), bat2d,
                     lw1, lb1.reshape(1, -1), lw2, lb2.reshape(1, -1))


# speedup vs baseline: 37.0105x; 1.2753x over previous
import functools

import jax
import jax.numpy as jnp
from jax import lax
from jax.experimental import pallas as pl
from jax.experimental.pallas import tpu as pltpu
from jax.experimental.pallas import tpu_sc as plsc

_N = 10000
_NPAD = 10240
_NG = 64
_NC = 2
_NS = 16
_NW = _NC * _NS
_CHUNK = 128
_NBUF = 8
_ROWS_PT = _NPAD // _NS
_BR = 1024


def _sc_params():
  return pltpu.CompilerParams(use_tc_tiling_on_sc=False)


def _sc_mesh():
  return plsc.VectorSubcoreMesh(core_axis_name="c", subcore_axis_name="s")


@functools.lru_cache(maxsize=None)
def _deg_kernel(rounds: int):
  def body(dst_hbm, ones_hbm, zeros_hbm, out_hbm, dst_v, ones_v, acc_sh, sem):
    c = lax.axis_index("c")
    s = lax.axis_index("s")
    wid = c * _NS + s
    pltpu.sync_copy(dst_hbm.at[wid], dst_v)
    pltpu.sync_copy(ones_hbm, ones_v)
    pltpu.sync_copy(zeros_hbm, acc_sh.at[pl.ds(s * _ROWS_PT, _ROWS_PT)])
    plsc.subcore_barrier()

    def scat(j, carry):
      pltpu.async_copy(ones_v, acc_sh.at[dst_v.at[j]], sem, add=True)
      return carry

    lax.fori_loop(0, rounds, scat, 0)

    def drain(j, carry):
      pltpu.make_async_copy(ones_v, acc_sh.at[dst_v.at[j]], sem).wait()
      return carry

    lax.fori_loop(0, rounds, drain, 0)
    plsc.subcore_barrier()
    pltpu.sync_copy(acc_sh.at[pl.ds(s * _ROWS_PT, _ROWS_PT)],
                    out_hbm.at[c, pl.ds(s * _ROWS_PT, _ROWS_PT)])

  return pl.kernel(
      body,
      out_type=jax.ShapeDtypeStruct((_NC, _NPAD), jnp.float32),
      mesh=_sc_mesh(),
      scratch_types=[
          pltpu.VMEM((rounds, _CHUNK), jnp.int32),
          pltpu.VMEM((_CHUNK,), jnp.float32),
          pltpu.VMEM_SHARED((_NPAD,), jnp.float32),
          pltpu.SemaphoreType.DMA,
      ],
      compiler_params=_sc_params(),
  )


@functools.lru_cache(maxsize=None)
def _agg_kernel(fh: int, rounds: int):
  ngroups = rounds // _NBUF
  assert ngroups % 2 == 0

  def body(g_hbm, src_hbm, dst_hbm, zeros_hbm, out_hbm,
           sidx, didx, buf_v, acc_sh, isem, gsem, ssem):
    c = lax.axis_index("c")
    s = lax.axis_index("s")

    pltpu.sync_copy(zeros_hbm, acc_sh.at[pl.ds(s * _ROWS_PT, _ROWS_PT)])
    pltpu.async_copy(src_hbm.at[c, s, pl.ds(0, _NBUF)], sidx.at[0],
                     isem.at[0, 0])
    pltpu.async_copy(dst_hbm.at[s, pl.ds(0, _NBUF)], didx.at[0],
                     isem.at[0, 1])
    plsc.subcore_barrier()

    def pair(hj, carry):
      for par in range(2):
        gi = 2 * hj + par
        base = gi * _NBUF

        pltpu.make_async_copy(src_hbm.at[c, s, pl.ds(base, _NBUF)],
                              sidx.at[par], isem.at[par, 0]).wait()
        pltpu.make_async_copy(dst_hbm.at[s, pl.ds(base, _NBUF)],
                              didx.at[par], isem.at[par, 1]).wait()

        for b in range(_NBUF):
          @pl.when(gi > 0)
          def _drain_prev():
            pltpu.make_async_copy(buf_v.at[b], acc_sh.at[didx.at[1 - par, b]],
                                  ssem.at[b]).wait()

          pltpu.async_copy(g_hbm.at[sidx.at[par, b]], buf_v.at[b],
                           gsem.at[b])

        @pl.when(gi + 1 < ngroups)
        def _prefetch():
          nb = (gi + 1) * _NBUF
          pltpu.async_copy(src_hbm.at[c, s, pl.ds(nb, _NBUF)],
                           sidx.at[1 - par], isem.at[1 - par, 0])
          pltpu.async_copy(dst_hbm.at[s, pl.ds(nb, _NBUF)],
                           didx.at[1 - par], isem.at[1 - par, 1])

        for b in range(_NBUF):
          pltpu.make_async_copy(g_hbm.at[sidx.at[par, b]], buf_v.at[b],
                                gsem.at[b]).wait()
          pltpu.async_copy(buf_v.at[b], acc_sh.at[didx.at[par, b]],
                           ssem.at[b], add=True)
      return carry

    lax.fori_loop(0, ngroups // 2, pair, 0)
    for b in range(_NBUF):
      pltpu.make_async_copy(buf_v.at[b], acc_sh.at[didx.at[1, b]],
                            ssem.at[b]).wait()
    plsc.subcore_barrier()
    pltpu.sync_copy(acc_sh.at[pl.ds(s * _ROWS_PT, _ROWS_PT)],
                    out_hbm.at[c, pl.ds(s * _ROWS_PT, _ROWS_PT)])

  return pl.kernel(
      body,
      out_type=jax.ShapeDtypeStruct((_NC, _NPAD, fh), jnp.float32),
      mesh=_sc_mesh(),
      scratch_types=[
          pltpu.VMEM((2, _NBUF, _CHUNK), jnp.int32),
          pltpu.VMEM((2, _NBUF, _CHUNK), jnp.int32),
          pltpu.VMEM((_NBUF, _CHUNK, fh), jnp.float32),
          pltpu.VMEM_SHARED((_NPAD, fh), jnp.float32),
          pltpu.SemaphoreType.DMA((2, 2)),
          pltpu.SemaphoreType.DMA((_NBUF,)),
          pltpu.SemaphoreType.DMA((_NBUF,)),
      ],
      compiler_params=_sc_params(),
  )


def _xw_call(x_pad, w1):
  nb = _NPAD // _BR
  fo = w1.shape[1]

  def body(x_ref, w_ref, u_ref):
    u_ref[...] = jnp.dot(x_ref[...], w_ref[...],
                         preferred_element_type=jnp.float32)

  return pl.pallas_call(
      body,
      grid=(nb,),
      in_specs=[
          pl.BlockSpec((_BR, 128), lambda i: (i, 0)),
          pl.BlockSpec((128, fo), lambda i: (0, 0)),
      ],
      out_specs=pl.BlockSpec((_BR, fo), lambda i: (i, 0)),
      out_shape=jax.ShapeDtypeStruct((_NPAD, fo), jnp.float32),
  )(x_pad, w1)


def _prep_call(deg_parts, u):
  nb = _NPAD // _BR
  fo = u.shape[1]
  fh = fo // 2

  def body(deg_ref, u_ref, dinv_ref, g_ref):
    i = pl.program_id(0)
    d = deg_ref[0] + deg_ref[1] + 1.0
    row = lax.broadcasted_iota(jnp.int32, (_BR, 1), 0) + i * _BR
    dinv = jnp.where(row < _N, lax.rsqrt(d), 0.0)
    dinv_ref[...] = dinv
    g = dinv * u_ref[...]
    g_ref[0] = g[:, :fh]
    g_ref[1] = g[:, fh:]

  return pl.pallas_call(
      body,
      grid=(nb,),
      in_specs=[
          pl.BlockSpec((_NC, _BR, 1), lambda i: (0, i, 0)),
          pl.BlockSpec((_BR, fo), lambda i: (i, 0)),
      ],
      out_specs=[
          pl.BlockSpec((_BR, 1), lambda i: (i, 0)),
          pl.BlockSpec((_NC, _BR, fh), lambda i: (0, i, 0)),
      ],
      out_shape=[
          jax.ShapeDtypeStruct((_NPAD, 1), jnp.float32),
          jax.ShapeDtypeStruct((_NC, _NPAD, fh), jnp.float32),
      ],
  )(deg_parts, u)


def _mid_call(parts, g, dinv, b2d, w):
  fh = g.shape[2]
  fo2 = w.shape[1]
  fh2 = fo2 // 2
  nb = _NPAD // _BR

  def body(p_ref, g_ref, dinv_ref, b_ref, w_ref, o_ref):
    agg = jnp.concatenate([p_ref[0] + g_ref[0], p_ref[1] + g_ref[1]], axis=1)
    h = jnp.maximum(dinv_ref[...] * agg + b_ref[...], 0.0)
    gn = dinv_ref[...] * jnp.dot(h, w_ref[...],
                                 preferred_element_type=jnp.float32)
    o_ref[0] = gn[:, :fh2]
    o_ref[1] = gn[:, fh2:]

  return pl.pallas_call(
      body,
      grid=(nb,),
      in_specs=[
          pl.BlockSpec((_NC, _BR, fh), lambda i: (0, i, 0)),
          pl.BlockSpec((_NC, _BR, fh), lambda i: (0, i, 0)),
          pl.BlockSpec((_BR, 1), lambda i: (i, 0)),
          pl.BlockSpec((1, 2 * fh), lambda i: (0, 0)),
          pl.BlockSpec((2 * fh, fo2), lambda i: (0, 0)),
      ],
      out_specs=pl.BlockSpec((_NC, _BR, fh2), lambda i: (0, i, 0)),
      out_shape=jax.ShapeDtypeStruct((_NC, _NPAD, fh2), jnp.float32),
  )(parts, g, dinv, b2d, w)


def _final_call(parts, g, dinv, b2d, bat2d, lw1, lb1, lw2, lb2):
  nb = _NPAD // _BR
  fh = g.shape[2]

  def body(p_ref, g_ref, dinv_ref, b_ref, bat_ref, lw1_ref, lb1_ref, lw2_ref,
           lb2_ref, o_ref, acc_ref):
    i = pl.program_id(0)

    @pl.when(i == 0)
    def _init():
      acc_ref[...] = jnp.full((_NG, 128), -jnp.inf, jnp.float32)

    agg = jnp.concatenate([p_ref[0] + g_ref[0], p_ref[1] + g_ref[1]], axis=1)
    h = jnp.maximum(dinv_ref[...] * agg + b_ref[...], 0.0)
    row = lax.broadcasted_iota(jnp.int32, (_BR, 1), 0) + i * _BR
    valid = row < _N
    bat = bat_ref[...]
    bmin = jnp.min(jnp.where(valid, bat, _NG - 1))
    bmax = jnp.max(jnp.where(valid, bat, 0))
    gcol = lax.broadcasted_iota(jnp.int32, (_NG, 1), 0)

    def gbody(gg, carry):
      m = (bat == gg) & valid
      red = jnp.max(jnp.where(m, h, -jnp.inf), axis=0, keepdims=True)
      acc_ref[...] = jnp.maximum(acc_ref[...],
                                 jnp.where(gcol == gg, red, -jnp.inf))
      return carry

    lax.fori_loop(bmin, bmax + 1, gbody, 0)

    @pl.when(i == nb - 1)
    def _fin():
      z = jnp.maximum(
          jnp.dot(acc_ref[...], lw1_ref[...],
                  preferred_element_type=jnp.float32) + lb1_ref[...], 0.0)
      o_ref[...] = jnp.dot(z, lw2_ref[...],
                           preferred_element_type=jnp.float32) + lb2_ref[...]

  return pl.pallas_call(
      body,
      grid=(nb,),
      in_specs=[
          pl.BlockSpec((_NC, _BR, fh), lambda i: (0, i, 0)),
          pl.BlockSpec((_NC, _BR, fh), lambda i: (0, i, 0)),
          pl.BlockSpec((_BR, 1), lambda i: (i, 0)),
          pl.BlockSpec((1, 128), lambda i: (0, 0)),
          pl.BlockSpec((_BR, 1), lambda i: (i, 0)),
          pl.BlockSpec((128, 64), lambda i: (0, 0)),
          pl.BlockSpec((1, 64), lambda i: (0, 0)),
          pl.BlockSpec((64, 10), lambda i: (0, 0)),
          pl.BlockSpec((1, 10), lambda i: (0, 0)),
      ],
      out_specs=pl.BlockSpec((_NG, 10), lambda i: (0, 0)),
      out_shape=jax.ShapeDtypeStruct((_NG, 10), jnp.float32),
      scratch_shapes=[pltpu.VMEM((_NG, 128), jnp.float32)],
  )(parts, g, dinv, b2d, bat2d, lw1, lb1, lw2, lb2)


def _pad_idx(n):
  return _N + (jnp.arange(n, dtype=jnp.int32) % (_NPAD - _N))


def kernel(x, edge_index, batch, W1, b1, W2, b2, W3, b3, W4, b4, W5, b5,
           W6, b6, lw1, lb1, lw2, lb2):
  e = edge_index.shape[1]
  src = edge_index[0].astype(jnp.int32)
  dst = edge_index[1].astype(jnp.int32)

  r_dg = -(-e // (_NW * _CHUNK))
  pad_dg = _pad_idx(_NW * r_dg * _CHUNK - e)
  dst_dg = jnp.concatenate([dst, pad_dg]).reshape(_NW, r_dg, _CHUNK)

  r_ag = -(-e // (_NS * _CHUNK))
  r_ag = -(-r_ag // (2 * _NBUF)) * (2 * _NBUF)
  pad_ag = _pad_idx(_NS * r_ag * _CHUNK - e)
  src_t = jnp.concatenate([src, pad_ag]).reshape(_NS, r_ag, _CHUNK)
  dst_t = jnp.concatenate([dst, pad_ag]).reshape(_NS, r_ag, _CHUNK)
  srcp = jnp.stack([src_t, src_t + _NPAD])

  x_pad = jnp.pad(x, ((0, _NPAD - _N), (0, 0)))
  bat2d = jnp.pad(batch.astype(jnp.int32), (0, _NPAD - _N),
                  constant_values=_NG - 1).reshape(_NPAD, 1)
  ones_c = jnp.ones((_CHUNK,), jnp.float32)
  zeros_r = jnp.zeros((_ROWS_PT,), jnp.float32)

  u1 = _xw_call(x_pad, W1)
  deg_parts = _deg_kernel(r_dg)(dst_dg, ones_c, zeros_r)
  dinv, g = _prep_call(deg_parts.reshape(_NC, _NPAD, 1), u1)

  ws = [W2, W3, W4, W5, W6]
  bs = [b1, b2, b3, b4, b5]
  parts = None
  for l in range(6):
    fh = g.shape[2]
    zeros_z = jnp.zeros((_ROWS_PT, fh), jnp.float32)
    parts = _agg_kernel(fh, r_ag)(g.reshape(_NC * _NPAD, fh), srcp, dst_t,
                                  zeros_z)
    if l < 5:
      g = _mid_call(parts, g, dinv, bs[l].reshape(1, -1), ws[l])

  return _final_call(parts, g, dinv, b6.reshape(1, -1), bat2d,
                     lw1, lb1.reshape(1, -1), lw2, lb2.reshape(1, -1))


# speedup vs baseline: 37.2057x; 1.0053x over previous
import functools

import jax
import jax.numpy as jnp
from jax import lax
from jax.experimental import pallas as pl
from jax.experimental.pallas import tpu as pltpu
from jax.experimental.pallas import tpu_sc as plsc

_N = 10000
_NPAD = 10240
_NG = 64
_NC = 2
_NS = 16
_NW = _NC * _NS
_CHUNK = 128
_NBUF = 8
_ROWS_PT = _NPAD // _NS
_BR = 2048


def _sc_params():
  return pltpu.CompilerParams(use_tc_tiling_on_sc=False)


def _sc_mesh():
  return plsc.VectorSubcoreMesh(core_axis_name="c", subcore_axis_name="s")


@functools.lru_cache(maxsize=None)
def _deg_kernel(rounds: int):
  def body(dst_hbm, ones_hbm, zeros_hbm, out_hbm, dst_v, ones_v, acc_sh, sem):
    c = lax.axis_index("c")
    s = lax.axis_index("s")
    wid = c * _NS + s
    pltpu.sync_copy(dst_hbm.at[wid], dst_v)
    pltpu.sync_copy(ones_hbm, ones_v)
    pltpu.sync_copy(zeros_hbm, acc_sh.at[pl.ds(s * _ROWS_PT, _ROWS_PT)])
    plsc.subcore_barrier()

    def scat(j, carry):
      pltpu.async_copy(ones_v, acc_sh.at[dst_v.at[j]], sem, add=True)
      return carry

    lax.fori_loop(0, rounds, scat, 0)

    def drain(j, carry):
      pltpu.make_async_copy(ones_v, acc_sh.at[dst_v.at[j]], sem).wait()
      return carry

    lax.fori_loop(0, rounds, drain, 0)
    plsc.subcore_barrier()
    pltpu.sync_copy(acc_sh.at[pl.ds(s * _ROWS_PT, _ROWS_PT)],
                    out_hbm.at[c, pl.ds(s * _ROWS_PT, _ROWS_PT)])

  return pl.kernel(
      body,
      out_type=jax.ShapeDtypeStruct((_NC, _NPAD), jnp.float32),
      mesh=_sc_mesh(),
      scratch_types=[
          pltpu.VMEM((rounds, _CHUNK), jnp.int32),
          pltpu.VMEM((_CHUNK,), jnp.float32),
          pltpu.VMEM_SHARED((_NPAD,), jnp.float32),
          pltpu.SemaphoreType.DMA,
      ],
      compiler_params=_sc_params(),
  )


@functools.lru_cache(maxsize=None)
def _agg_kernel(fh: int, rounds: int):
  nbuf = _NBUF
  ngroups = rounds // nbuf
  assert ngroups % 2 == 0

  def body(g_hbm, src_hbm, dst_hbm, zeros_hbm, out_hbm,
           sidx, didx, buf_v, acc_sh, isem, gsem, ssem):
    c = lax.axis_index("c")
    s = lax.axis_index("s")

    pltpu.sync_copy(zeros_hbm, acc_sh.at[pl.ds(s * _ROWS_PT, _ROWS_PT)])
    pltpu.async_copy(src_hbm.at[c, s, pl.ds(0, nbuf)], sidx.at[0],
                     isem.at[0, 0])
    pltpu.async_copy(dst_hbm.at[s, pl.ds(0, nbuf)], didx.at[0],
                     isem.at[0, 1])
    plsc.subcore_barrier()

    def pair(hj, carry):
      for par in range(2):
        gi = 2 * hj + par
        base = gi * nbuf

        pltpu.make_async_copy(src_hbm.at[c, s, pl.ds(base, nbuf)],
                              sidx.at[par], isem.at[par, 0]).wait()
        pltpu.make_async_copy(dst_hbm.at[s, pl.ds(base, nbuf)],
                              didx.at[par], isem.at[par, 1]).wait()

        for b in range(nbuf):
          @pl.when(gi > 0)
          def _drain_prev():
            pltpu.make_async_copy(buf_v.at[b], acc_sh.at[didx.at[1 - par, b]],
                                  ssem.at[b]).wait()

          pltpu.async_copy(g_hbm.at[sidx.at[par, b]], buf_v.at[b],
                           gsem.at[b])

        @pl.when(gi + 1 < ngroups)
        def _prefetch():
          nb = (gi + 1) * nbuf
          pltpu.async_copy(src_hbm.at[c, s, pl.ds(nb, nbuf)],
                           sidx.at[1 - par], isem.at[1 - par, 0])
          pltpu.async_copy(dst_hbm.at[s, pl.ds(nb, nbuf)],
                           didx.at[1 - par], isem.at[1 - par, 1])

        for b in range(nbuf):
          pltpu.make_async_copy(g_hbm.at[sidx.at[par, b]], buf_v.at[b],
                                gsem.at[b]).wait()
          pltpu.async_copy(buf_v.at[b], acc_sh.at[didx.at[par, b]],
                           ssem.at[b], add=True)
      return carry

    lax.fori_loop(0, ngroups // 2, pair, 0)
    for b in range(nbuf):
      pltpu.make_async_copy(buf_v.at[b], acc_sh.at[didx.at[1, b]],
                            ssem.at[b]).wait()
    plsc.subcore_barrier()
    pltpu.sync_copy(acc_sh.at[pl.ds(s * _ROWS_PT, _ROWS_PT)],
                    out_hbm.at[c, pl.ds(s * _ROWS_PT, _ROWS_PT)])

  return pl.kernel(
      body,
      out_type=jax.ShapeDtypeStruct((_NC, _NPAD, fh), jnp.float32),
      mesh=_sc_mesh(),
      scratch_types=[
          pltpu.VMEM((2, nbuf, _CHUNK), jnp.int32),
          pltpu.VMEM((2, nbuf, _CHUNK), jnp.int32),
          pltpu.VMEM((nbuf, _CHUNK, fh), jnp.float32),
          pltpu.VMEM_SHARED((_NPAD, fh), jnp.float32),
          pltpu.SemaphoreType.DMA((2, 2)),
          pltpu.SemaphoreType.DMA((nbuf,)),
          pltpu.SemaphoreType.DMA((nbuf,)),
      ],
      compiler_params=_sc_params(),
  )


def _xw_call(x_pad, w1):
  nb = _NPAD // _BR
  fo = w1.shape[1]

  def body(x_ref, w_ref, u_ref):
    u_ref[...] = jnp.dot(x_ref[...], w_ref[...],
                         preferred_element_type=jnp.float32)

  return pl.pallas_call(
      body,
      grid=(nb,),
      in_specs=[
          pl.BlockSpec((_BR, 128), lambda i: (i, 0)),
          pl.BlockSpec((128, fo), lambda i: (0, 0)),
      ],
      out_specs=pl.BlockSpec((_BR, fo), lambda i: (i, 0)),
      out_shape=jax.ShapeDtypeStruct((_NPAD, fo), jnp.float32),
  )(x_pad, w1)


def _prep_call(deg_parts, u):
  nb = _NPAD // _BR
  fo = u.shape[1]
  fh = fo // 2

  def body(deg_ref, u_ref, dinv_ref, g_ref):
    i = pl.program_id(0)
    d = deg_ref[0] + deg_ref[1] + 1.0
    row = lax.broadcasted_iota(jnp.int32, (_BR, 1), 0) + i * _BR
    dinv = jnp.where(row < _N, lax.rsqrt(d), 0.0)
    dinv_ref[...] = jnp.broadcast_to(dinv, (_BR, 128))
    g = dinv * u_ref[...]
    g_ref[0] = g[:, :fh]
    g_ref[1] = g[:, fh:]

  return pl.pallas_call(
      body,
      grid=(nb,),
      in_specs=[
          pl.BlockSpec((_NC, _BR, 1), lambda i: (0, i, 0)),
          pl.BlockSpec((_BR, fo), lambda i: (i, 0)),
      ],
      out_specs=[
          pl.BlockSpec((_BR, 128), lambda i: (i, 0)),
          pl.BlockSpec((_NC, _BR, fh), lambda i: (0, i, 0)),
      ],
      out_shape=[
          jax.ShapeDtypeStruct((_NPAD, 128), jnp.float32),
          jax.ShapeDtypeStruct((_NC, _NPAD, fh), jnp.float32),
      ],
  )(deg_parts, u)


def _mid_call(parts, g, dinv, b2d, w):
  fh = g.shape[2]
  fo2 = w.shape[1]
  fh2 = fo2 // 2
  nb = _NPAD // _BR

  def body(p_ref, g_ref, dinv_ref, b_ref, w_ref, o_ref):
    agg = jnp.concatenate([p_ref[0] + g_ref[0], p_ref[1] + g_ref[1]], axis=1)
    dv = dinv_ref[...]
    h = jnp.maximum(dv[:, :2 * fh] * agg + b_ref[...], 0.0)
    gn = dv[:, :fo2] * jnp.dot(h, w_ref[...],
                               preferred_element_type=jnp.float32)
    o_ref[0] = gn[:, :fh2]
    o_ref[1] = gn[:, fh2:]

  return pl.pallas_call(
      body,
      grid=(nb,),
      in_specs=[
          pl.BlockSpec((_NC, _BR, fh), lambda i: (0, i, 0)),
          pl.BlockSpec((_NC, _BR, fh), lambda i: (0, i, 0)),
          pl.BlockSpec((_BR, 128), lambda i: (i, 0)),
          pl.BlockSpec((1, 2 * fh), lambda i: (0, 0)),
          pl.BlockSpec((2 * fh, fo2), lambda i: (0, 0)),
      ],
      out_specs=pl.BlockSpec((_NC, _BR, fh2), lambda i: (0, i, 0)),
      out_shape=jax.ShapeDtypeStruct((_NC, _NPAD, fh2), jnp.float32),
  )(parts, g, dinv, b2d, w)


def _final_call(parts, g, dinv, b2d, bat2d, lw1, lb1, lw2, lb2):
  nb = _NPAD // _BR
  fh = g.shape[2]

  def body(p_ref, g_ref, dinv_ref, b_ref, bat_ref, lw1_ref, lb1_ref, lw2_ref,
           lb2_ref, o_ref, acc_ref):
    i = pl.program_id(0)

    @pl.when(i == 0)
    def _init():
      acc_ref[...] = jnp.full((_NG, 128), -jnp.inf, jnp.float32)

    agg = jnp.concatenate([p_ref[0] + g_ref[0], p_ref[1] + g_ref[1]], axis=1)
    h = jnp.maximum(dinv_ref[...] * agg + b_ref[...], 0.0)
    row = lax.broadcasted_iota(jnp.int32, (_BR, 1), 0) + i * _BR
    valid = row < _N
    bat = bat_ref[...]
    bmin = jnp.min(jnp.where(valid, bat, _NG - 1))
    bmax = jnp.max(jnp.where(valid, bat, 0))
    gcol = lax.broadcasted_iota(jnp.int32, (_NG, 1), 0)

    def gbody(gg, carry):
      m = (bat == gg) & valid
      red = jnp.max(jnp.where(m, h, -jnp.inf), axis=0, keepdims=True)
      acc_ref[...] = jnp.maximum(acc_ref[...],
                                 jnp.where(gcol == gg, red, -jnp.inf))
      return carry

    lax.fori_loop(bmin, bmax + 1, gbody, 0)

    @pl.when(i == nb - 1)
    def _fin():
      z = jnp.maximum(
          jnp.dot(acc_ref[...], lw1_ref[...],
                  preferred_element_type=jnp.float32) + lb1_ref[...], 0.0)
      o_ref[...] = jnp.dot(z, lw2_ref[...],
                           preferred_element_type=jnp.float32) + lb2_ref[...]

  return pl.pallas_call(
      body,
      grid=(nb,),
      in_specs=[
          pl.BlockSpec((_NC, _BR, fh), lambda i: (0, i, 0)),
          pl.BlockSpec((_NC, _BR, fh), lambda i: (0, i, 0)),
          pl.BlockSpec((_BR, 128), lambda i: (i, 0)),
          pl.BlockSpec((1, 128), lambda i: (0, 0)),
          pl.BlockSpec((_BR, 1), lambda i: (i, 0)),
          pl.BlockSpec((128, 64), lambda i: (0, 0)),
          pl.BlockSpec((1, 64), lambda i: (0, 0)),
          pl.BlockSpec((64, 10), lambda i: (0, 0)),
          pl.BlockSpec((1, 10), lambda i: (0, 0)),
      ],
      out_specs=pl.BlockSpec((_NG, 10), lambda i: (0, 0)),
      out_shape=jax.ShapeDtypeStruct((_NG, 10), jnp.float32),
      scratch_shapes=[pltpu.VMEM((_NG, 128), jnp.float32)],
  )(parts, g, dinv, b2d, bat2d, lw1, lb1, lw2, lb2)


def _pad_idx(n):
  return _N + (jnp.arange(n, dtype=jnp.int32) % (_NPAD - _N))


def kernel(x, edge_index, batch, W1, b1, W2, b2, W3, b3, W4, b4, W5, b5,
           W6, b6, lw1, lb1, lw2, lb2):
  e = edge_index.shape[1]
  src = edge_index[0].astype(jnp.int32)
  dst = edge_index[1].astype(jnp.int32)

  r_dg = -(-e // (_NW * _CHUNK))
  pad_dg = _pad_idx(_NW * r_dg * _CHUNK - e)
  dst_dg = jnp.concatenate([dst, pad_dg]).reshape(_NW, r_dg, _CHUNK)

  r_ag = -(-e // (_NS * _CHUNK))
  r_ag = -(-r_ag // 32) * 32
  pad_ag = _pad_idx(_NS * r_ag * _CHUNK - e)
  src_t = jnp.concatenate([src, pad_ag]).reshape(_NS, r_ag, _CHUNK)
  dst_t = jnp.concatenate([dst, pad_ag]).reshape(_NS, r_ag, _CHUNK)
  srcp = jnp.stack([src_t, src_t + _NPAD])

  x_pad = jnp.pad(x, ((0, _NPAD - _N), (0, 0)))
  bat2d = jnp.pad(batch.astype(jnp.int32), (0, _NPAD - _N),
                  constant_values=_NG - 1).reshape(_NPAD, 1)
  ones_c = jnp.ones((_CHUNK,), jnp.float32)
  zeros_r = jnp.zeros((_ROWS_PT,), jnp.float32)

  u1 = _xw_call(x_pad, W1)
  deg_parts = _deg_kernel(r_dg)(dst_dg, ones_c, zeros_r)
  dinv, g = _prep_call(deg_parts.reshape(_NC, _NPAD, 1), u1)

  ws = [W2, W3, W4, W5, W6]
  bs = [b1, b2, b3, b4, b5]
  parts = None
  for l in range(6):
    fh = g.shape[2]
    zeros_z = jnp.zeros((_ROWS_PT, fh), jnp.float32)
    parts = _agg_kernel(fh, r_ag)(g.reshape(_NC * _NPAD, fh), srcp, dst_t,
                                  zeros_z)
    if l < 5:
      g = _mid_call(parts, g, dinv, bs[l].reshape(1, -1), ws[l])

  return _final_call(parts, g, dinv, b6.reshape(1, -1), bat2d,
                     lw1, lb1.reshape(1, -1), lw2, lb2.reshape(1, -1))


# speedup vs baseline: 39.5028x; 1.0617x over previous
import functools

import jax
import jax.numpy as jnp
from jax import lax
from jax.experimental import pallas as pl
from jax.experimental.pallas import tpu as pltpu
from jax.experimental.pallas import tpu_sc as plsc

_N = 10000
_NPAD = 10240
_NG = 64
_NC = 2
_NS = 16
_NW = _NC * _NS
_CHUNK = 128
_NBUF = 8
_ROWS_PT = _NPAD // _NS
_BR = 2048


def _sc_params():
  return pltpu.CompilerParams(use_tc_tiling_on_sc=False)


def _sc_mesh():
  return plsc.VectorSubcoreMesh(core_axis_name="c", subcore_axis_name="s")


@functools.lru_cache(maxsize=None)
def _deg_kernel(rounds: int):
  def body(dst_hbm, ones_hbm, zeros_hbm, out_hbm, dst_v, ones_v, acc_sh, sem):
    c = lax.axis_index("c")
    s = lax.axis_index("s")
    wid = c * _NS + s
    pltpu.sync_copy(dst_hbm.at[wid], dst_v)
    pltpu.sync_copy(ones_hbm, ones_v)
    pltpu.sync_copy(zeros_hbm, acc_sh.at[pl.ds(s * _ROWS_PT, _ROWS_PT)])
    plsc.subcore_barrier()

    def scat(j, carry):
      pltpu.async_copy(ones_v, acc_sh.at[dst_v.at[j]], sem, add=True)
      return carry

    lax.fori_loop(0, rounds, scat, 0)

    def drain(j, carry):
      pltpu.make_async_copy(ones_v, acc_sh.at[dst_v.at[j]], sem).wait()
      return carry

    lax.fori_loop(0, rounds, drain, 0)
    plsc.subcore_barrier()
    pltpu.sync_copy(acc_sh.at[pl.ds(s * _ROWS_PT, _ROWS_PT)],
                    out_hbm.at[c, pl.ds(s * _ROWS_PT, _ROWS_PT)])

  return pl.kernel(
      body,
      out_type=jax.ShapeDtypeStruct((_NC, _NPAD), jnp.float32),
      mesh=_sc_mesh(),
      scratch_types=[
          pltpu.VMEM((rounds, _CHUNK), jnp.int32),
          pltpu.VMEM((_CHUNK,), jnp.float32),
          pltpu.VMEM_SHARED((_NPAD,), jnp.float32),
          pltpu.SemaphoreType.DMA,
      ],
      compiler_params=_sc_params(),
  )


@functools.lru_cache(maxsize=None)
def _agg_kernel(fh: int, rounds: int):
  nbuf = _NBUF
  ngroups = rounds // nbuf
  assert ngroups % 2 == 0

  def body(g_hbm, src_hbm, dst_hbm, zeros_hbm, out_hbm,
           sidx, didx, buf_v, acc_sh, isem, gsem, ssem):
    c = lax.axis_index("c")
    s = lax.axis_index("s")

    pltpu.sync_copy(zeros_hbm, acc_sh.at[pl.ds(s * _ROWS_PT, _ROWS_PT)])
    pltpu.async_copy(src_hbm.at[c, s, pl.ds(0, nbuf)], sidx.at[0],
                     isem.at[0, 0])
    pltpu.async_copy(dst_hbm.at[c, s, pl.ds(0, nbuf)], didx.at[0],
                     isem.at[0, 1])
    plsc.subcore_barrier()

    def pair(hj, carry):
      for par in range(2):
        gi = 2 * hj + par
        base = gi * nbuf

        pltpu.make_async_copy(src_hbm.at[c, s, pl.ds(base, nbuf)],
                              sidx.at[par], isem.at[par, 0]).wait()
        pltpu.make_async_copy(dst_hbm.at[c, s, pl.ds(base, nbuf)],
                              didx.at[par], isem.at[par, 1]).wait()

        for b in range(nbuf):
          @pl.when(gi > 0)
          def _drain_prev():
            pltpu.make_async_copy(buf_v.at[b], acc_sh.at[didx.at[1 - par, b]],
                                  ssem.at[b]).wait()

          pltpu.async_copy(g_hbm.at[sidx.at[par, b]], buf_v.at[b],
                           gsem.at[b])

        @pl.when(gi + 1 < ngroups)
        def _prefetch():
          nb = (gi + 1) * nbuf
          pltpu.async_copy(src_hbm.at[c, s, pl.ds(nb, nbuf)],
                           sidx.at[1 - par], isem.at[1 - par, 0])
          pltpu.async_copy(dst_hbm.at[c, s, pl.ds(nb, nbuf)],
                           didx.at[1 - par], isem.at[1 - par, 1])

        for b in range(nbuf):
          pltpu.make_async_copy(g_hbm.at[sidx.at[par, b]], buf_v.at[b],
                                gsem.at[b]).wait()
          pltpu.async_copy(buf_v.at[b], acc_sh.at[didx.at[par, b]],
                           ssem.at[b], add=True)
      return carry

    lax.fori_loop(0, ngroups // 2, pair, 0)
    for b in range(nbuf):
      pltpu.make_async_copy(buf_v.at[b], acc_sh.at[didx.at[1, b]],
                            ssem.at[b]).wait()
    plsc.subcore_barrier()
    pltpu.sync_copy(acc_sh.at[pl.ds(s * _ROWS_PT, _ROWS_PT)],
                    out_hbm.at[c, pl.ds(s * _ROWS_PT, _ROWS_PT)])

  return pl.kernel(
      body,
      out_type=jax.ShapeDtypeStruct((_NC, _NPAD, fh), jnp.float32),
      mesh=_sc_mesh(),
      scratch_types=[
          pltpu.VMEM((2, nbuf, _CHUNK), jnp.int32),
          pltpu.VMEM((2, nbuf, _CHUNK), jnp.int32),
          pltpu.VMEM((nbuf, _CHUNK, fh), jnp.float32),
          pltpu.VMEM_SHARED((_NPAD, fh), jnp.float32),
          pltpu.SemaphoreType.DMA((2, 2)),
          pltpu.SemaphoreType.DMA((nbuf,)),
          pltpu.SemaphoreType.DMA((nbuf,)),
      ],
      compiler_params=_sc_params(),
  )


def _xw_call(x_pad, w1):
  nb = _NPAD // _BR
  fo = w1.shape[1]

  def body(x_ref, w_ref, u_ref):
    u_ref[...] = jnp.dot(x_ref[...], w_ref[...],
                         preferred_element_type=jnp.float32)

  return pl.pallas_call(
      body,
      grid=(nb,),
      in_specs=[
          pl.BlockSpec((_BR, 128), lambda i: (i, 0)),
          pl.BlockSpec((128, fo), lambda i: (0, 0)),
      ],
      out_specs=pl.BlockSpec((_BR, fo), lambda i: (i, 0)),
      out_shape=jax.ShapeDtypeStruct((_NPAD, fo), jnp.float32),
  )(x_pad, w1)


def _prep_call(deg_parts, u):
  nb = _NPAD // _BR
  fo = u.shape[1]

  def body(deg_ref, u_ref, dinv_ref, g_ref):
    i = pl.program_id(0)
    d = deg_ref[0] + deg_ref[1] + 1.0
    row = lax.broadcasted_iota(jnp.int32, (_BR, 1), 0) + i * _BR
    dinv = jnp.where(row < _N, lax.rsqrt(d), 0.0)
    dinv_ref[...] = jnp.broadcast_to(dinv, (_BR, 128))
    g_ref[...] = dinv * u_ref[...]

  return pl.pallas_call(
      body,
      grid=(nb,),
      in_specs=[
          pl.BlockSpec((_NC, _BR, 1), lambda i: (0, i, 0)),
          pl.BlockSpec((_BR, fo), lambda i: (i, 0)),
      ],
      out_specs=[
          pl.BlockSpec((_BR, 128), lambda i: (i, 0)),
          pl.BlockSpec((_BR, fo), lambda i: (i, 0)),
      ],
      out_shape=[
          jax.ShapeDtypeStruct((_NPAD, 128), jnp.float32),
          jax.ShapeDtypeStruct((_NPAD, fo), jnp.float32),
      ],
  )(deg_parts, u)


def _mid_call(parts, g, dinv, b2d, w):
  fh = g.shape[2]
  fo2 = w.shape[1]
  fh2 = fo2 // 2
  nb = _NPAD // _BR

  def body(p_ref, g_ref, dinv_ref, b_ref, w_ref, o_ref):
    agg = jnp.concatenate([p_ref[0] + g_ref[0], p_ref[1] + g_ref[1]], axis=1)
    dv = dinv_ref[...]
    h = jnp.maximum(dv[:, :2 * fh] * agg + b_ref[...], 0.0)
    gn = dv[:, :fo2] * jnp.dot(h, w_ref[...],
                               preferred_element_type=jnp.float32)
    o_ref[0] = gn[:, :fh2]
    o_ref[1] = gn[:, fh2:]

  return pl.pallas_call(
      body,
      grid=(nb,),
      in_specs=[
          pl.BlockSpec((_NC, _BR, fh), lambda i: (0, i, 0)),
          pl.BlockSpec((_NC, _BR, fh), lambda i: (0, i, 0)),
          pl.BlockSpec((_BR, 128), lambda i: (i, 0)),
          pl.BlockSpec((1, 2 * fh), lambda i: (0, 0)),
          pl.BlockSpec((2 * fh, fo2), lambda i: (0, 0)),
      ],
      out_specs=pl.BlockSpec((_NC, _BR, fh2), lambda i: (0, i, 0)),
      out_shape=jax.ShapeDtypeStruct((_NC, _NPAD, fh2), jnp.float32),
  )(parts, g, dinv, b2d, w)


def _mid_es_call(parts, g, dinv, b2d, w, split_out):
  fo = g.shape[1]
  fo2 = w.shape[1]
  fh2 = fo2 // 2
  nb = _NPAD // _BR

  def body(p_ref, g_ref, dinv_ref, b_ref, w_ref, o_ref):
    agg = p_ref[0] + p_ref[1] + g_ref[...]
    dv = dinv_ref[...]
    h = jnp.maximum(dv[:, :fo] * agg + b_ref[...], 0.0)
    gn = dv[:, :fo2] * jnp.dot(h, w_ref[...],
                               preferred_element_type=jnp.float32)
    if split_out:
      o_ref[0] = gn[:, :fh2]
      o_ref[1] = gn[:, fh2:]
    else:
      o_ref[...] = gn

  if split_out:
    out_spec = pl.BlockSpec((_NC, _BR, fh2), lambda i: (0, i, 0))
    out_shape = jax.ShapeDtypeStruct((_NC, _NPAD, fh2), jnp.float32)
  else:
    out_spec = pl.BlockSpec((_BR, fo2), lambda i: (i, 0))
    out_shape = jax.ShapeDtypeStruct((_NPAD, fo2), jnp.float32)

  return pl.pallas_call(
      body,
      grid=(nb,),
      in_specs=[
          pl.BlockSpec((_NC, _BR, fo), lambda i: (0, i, 0)),
          pl.BlockSpec((_BR, fo), lambda i: (i, 0)),
          pl.BlockSpec((_BR, 128), lambda i: (i, 0)),
          pl.BlockSpec((1, fo), lambda i: (0, 0)),
          pl.BlockSpec((fo, fo2), lambda i: (0, 0)),
      ],
      out_specs=out_spec,
      out_shape=out_shape,
  )(parts, g, dinv, b2d, w)


def _final_call(parts, g, dinv, b2d, bat2d, lw1, lb1, lw2, lb2):
  nb = _NPAD // _BR
  fh = g.shape[2]

  def body(p_ref, g_ref, dinv_ref, b_ref, bat_ref, lw1_ref, lb1_ref, lw2_ref,
           lb2_ref, o_ref, acc_ref):
    i = pl.program_id(0)

    @pl.when(i == 0)
    def _init():
      acc_ref[...] = jnp.full((_NG, 128), -jnp.inf, jnp.float32)

    agg = jnp.concatenate([p_ref[0] + g_ref[0], p_ref[1] + g_ref[1]], axis=1)
    h = jnp.maximum(dinv_ref[...] * agg + b_ref[...], 0.0)
    row = lax.broadcasted_iota(jnp.int32, (_BR, 1), 0) + i * _BR
    valid = row < _N
    bat = bat_ref[...]
    bmin = jnp.min(jnp.where(valid, bat, _NG - 1))
    bmax = jnp.max(jnp.where(valid, bat, 0))
    gcol = lax.broadcasted_iota(jnp.int32, (_NG, 1), 0)

    def gbody(gg, carry):
      m = (bat == gg) & valid
      red = jnp.max(jnp.where(m, h, -jnp.inf), axis=0, keepdims=True)
      acc_ref[...] = jnp.maximum(acc_ref[...],
                                 jnp.where(gcol == gg, red, -jnp.inf))
      return carry

    lax.fori_loop(bmin, bmax + 1, gbody, 0)

    @pl.when(i == nb - 1)
    def _fin():
      z = jnp.maximum(
          jnp.dot(acc_ref[...], lw1_ref[...],
                  preferred_element_type=jnp.float32) + lb1_ref[...], 0.0)
      o_ref[...] = jnp.dot(z, lw2_ref[...],
                           preferred_element_type=jnp.float32) + lb2_ref[...]

  return pl.pallas_call(
      body,
      grid=(nb,),
      in_specs=[
          pl.BlockSpec((_NC, _BR, fh), lambda i: (0, i, 0)),
          pl.BlockSpec((_NC, _BR, fh), lambda i: (0, i, 0)),
          pl.BlockSpec((_BR, 128), lambda i: (i, 0)),
          pl.BlockSpec((1, 128), lambda i: (0, 0)),
          pl.BlockSpec((_BR, 1), lambda i: (i, 0)),
          pl.BlockSpec((128, 64), lambda i: (0, 0)),
          pl.BlockSpec((1, 64), lambda i: (0, 0)),
          pl.BlockSpec((64, 10), lambda i: (0, 0)),
          pl.BlockSpec((1, 10), lambda i: (0, 0)),
      ],
      out_specs=pl.BlockSpec((_NG, 10), lambda i: (0, 0)),
      out_shape=jax.ShapeDtypeStruct((_NG, 10), jnp.float32),
      scratch_shapes=[pltpu.VMEM((_NG, 128), jnp.float32)],
  )(parts, g, dinv, b2d, bat2d, lw1, lb1, lw2, lb2)


def _pad_idx(n):
  return _N + (jnp.arange(n, dtype=jnp.int32) % (_NPAD - _N))


def kernel(x, edge_index, batch, W1, b1, W2, b2, W3, b3, W4, b4, W5, b5,
           W6, b6, lw1, lb1, lw2, lb2):
  e = edge_index.shape[1]
  src = edge_index[0].astype(jnp.int32)
  dst = edge_index[1].astype(jnp.int32)

  r_es = -(-e // (_NW * _CHUNK))
  r_es = -(-r_es // (2 * _NBUF)) * (2 * _NBUF)
  pad_es = _pad_idx(_NW * r_es * _CHUNK - e)
  src_es = jnp.concatenate([src, pad_es]).reshape(_NC, _NS, r_es, _CHUNK)
  dst_es = jnp.concatenate([dst, pad_es]).reshape(_NC, _NS, r_es, _CHUNK)
  dst_dg = dst_es.reshape(_NW, r_es, _CHUNK)

  r_ag = -(-e // (_NS * _CHUNK))
  r_ag = -(-r_ag // (2 * _NBUF)) * (2 * _NBUF)
  pad_ag = _pad_idx(_NS * r_ag * _CHUNK - e)
  src_t = jnp.concatenate([src, pad_ag]).reshape(_NS, r_ag, _CHUNK)
  dst_t = jnp.concatenate([dst, pad_ag]).reshape(_NS, r_ag, _CHUNK)
  srcp = jnp.stack([src_t, src_t + _NPAD])
  dstp = jnp.stack([dst_t, dst_t])

  x_pad = jnp.pad(x, ((0, _NPAD - _N), (0, 0)))
  bat2d = jnp.pad(batch.astype(jnp.int32), (0, _NPAD - _N),
                  constant_values=_NG - 1).reshape(_NPAD, 1)
  ones_c = jnp.ones((_CHUNK,), jnp.float32)
  zeros_r = jnp.zeros((_ROWS_PT,), jnp.float32)

  u1 = _xw_call(x_pad, W1)
  deg_parts = _deg_kernel(r_es)(dst_dg, ones_c, zeros_r)
  dinv, g = _prep_call(deg_parts.reshape(_NC, _NPAD, 1), u1)

  parts = _agg_kernel(16, r_es)(g, src_es, dst_es,
                                jnp.zeros((_ROWS_PT, 16), jnp.float32))
  g = _mid_es_call(parts, g, dinv, b1.reshape(1, -1), W2, False)
  parts = _agg_kernel(32, r_es)(g, src_es, dst_es,
                                jnp.zeros((_ROWS_PT, 32), jnp.float32))
  g = _mid_es_call(parts, g, dinv, b2.reshape(1, -1), W3, True)

  ws = [W4, W5, W6]
  bs = [b3, b4, b5]
  for l in range(4):
    fh = g.shape[2]
    zeros_z = jnp.zeros((_ROWS_PT, fh), jnp.float32)
    parts = _agg_kernel(fh, r_ag)(g.reshape(_NC * _NPAD, fh), srcp, dstp,
                                  zeros_z)
    if l < 3:
      g = _mid_call(parts, g, dinv, bs[l].reshape(1, -1), ws[l])

  return _final_call(parts, g, dinv, b6.reshape(1, -1), bat2d,
                     lw1, lb1.reshape(1, -1), lw2, lb2.reshape(1, -1))


# speedup vs baseline: 40.3513x; 1.0215x over previous
import functools

import jax
import jax.numpy as jnp
from jax import lax
from jax.experimental import pallas as pl
from jax.experimental.pallas import tpu as pltpu
from jax.experimental.pallas import tpu_sc as plsc

_N = 10000
_NPAD = 10240
_NG = 64
_NC = 2
_NS = 16
_NW = _NC * _NS
_CHUNK = 128
_NBUF = 8
_ROWS_PT = _NPAD // _NS
_BR = 2048


def _sc_params():
  return pltpu.CompilerParams(use_tc_tiling_on_sc=False)


def _sc_mesh():
  return plsc.VectorSubcoreMesh(core_axis_name="c", subcore_axis_name="s")


@functools.lru_cache(maxsize=None)
def _deg_kernel(rounds: int):
  def body(dst_hbm, ones_hbm, zeros_hbm, out_hbm, dst_v, ones_v, acc_sh, sem):
    c = lax.axis_index("c")
    s = lax.axis_index("s")
    wid = c * _NS + s
    pltpu.sync_copy(dst_hbm.at[wid], dst_v)
    pltpu.sync_copy(ones_hbm, ones_v)
    pltpu.sync_copy(zeros_hbm, acc_sh.at[pl.ds(s * _ROWS_PT, _ROWS_PT)])
    plsc.subcore_barrier()

    def scat(j, carry):
      pltpu.async_copy(ones_v, acc_sh.at[dst_v.at[j]], sem, add=True)
      return carry

    lax.fori_loop(0, rounds, scat, 0)

    def drain(j, carry):
      pltpu.make_async_copy(ones_v, acc_sh.at[dst_v.at[j]], sem).wait()
      return carry

    lax.fori_loop(0, rounds, drain, 0)
    plsc.subcore_barrier()
    pltpu.sync_copy(acc_sh.at[pl.ds(s * _ROWS_PT, _ROWS_PT)],
                    out_hbm.at[c, pl.ds(s * _ROWS_PT, _ROWS_PT)])

  return pl.kernel(
      body,
      out_type=jax.ShapeDtypeStruct((_NC, _NPAD), jnp.float32),
      mesh=_sc_mesh(),
      scratch_types=[
          pltpu.VMEM((rounds, _CHUNK), jnp.int32),
          pltpu.VMEM((_CHUNK,), jnp.float32),
          pltpu.VMEM_SHARED((_NPAD,), jnp.float32),
          pltpu.SemaphoreType.DMA,
      ],
      compiler_params=_sc_params(),
  )


@functools.lru_cache(maxsize=None)
def _agg_kernel(fh: int, rounds: int):
  nbuf = _NBUF
  ngroups = rounds // nbuf
  assert ngroups % 2 == 0

  def body(g_hbm, src_hbm, dst_hbm, zeros_hbm, out_hbm,
           sidx, didx, buf_v, acc_sh, isem, gsem, ssem):
    c = lax.axis_index("c")
    s = lax.axis_index("s")

    pltpu.sync_copy(zeros_hbm, acc_sh.at[pl.ds(s * _ROWS_PT, _ROWS_PT)])
    pltpu.async_copy(src_hbm.at[c, s, pl.ds(0, nbuf)], sidx.at[0],
                     isem.at[0, 0])
    pltpu.async_copy(dst_hbm.at[c, s, pl.ds(0, nbuf)], didx.at[0],
                     isem.at[0, 1])
    plsc.subcore_barrier()

    def pair(hj, carry):
      for par in range(2):
        gi = 2 * hj + par
        base = gi * nbuf

        pltpu.make_async_copy(src_hbm.at[c, s, pl.ds(base, nbuf)],
                              sidx.at[par], isem.at[par, 0]).wait()
        pltpu.make_async_copy(dst_hbm.at[c, s, pl.ds(base, nbuf)],
                              didx.at[par], isem.at[par, 1]).wait()

        for b in range(nbuf):
          @pl.when(gi > 0)
          def _drain_prev():
            pltpu.make_async_copy(buf_v.at[b], acc_sh.at[didx.at[1 - par, b]],
                                  ssem.at[b]).wait()

          pltpu.async_copy(g_hbm.at[sidx.at[par, b]], buf_v.at[b],
                           gsem.at[b])

        @pl.when(gi + 1 < ngroups)
        def _prefetch():
          nb = (gi + 1) * nbuf
          pltpu.async_copy(src_hbm.at[c, s, pl.ds(nb, nbuf)],
                           sidx.at[1 - par], isem.at[1 - par, 0])
          pltpu.async_copy(dst_hbm.at[c, s, pl.ds(nb, nbuf)],
                           didx.at[1 - par], isem.at[1 - par, 1])

        for b in range(nbuf):
          pltpu.make_async_copy(g_hbm.at[sidx.at[par, b]], buf_v.at[b],
                                gsem.at[b]).wait()
          pltpu.async_copy(buf_v.at[b], acc_sh.at[didx.at[par, b]],
                           ssem.at[b], add=True)
      return carry

    lax.fori_loop(0, ngroups // 2, pair, 0)
    for b in range(nbuf):
      pltpu.make_async_copy(buf_v.at[b], acc_sh.at[didx.at[1, b]],
                            ssem.at[b]).wait()
    plsc.subcore_barrier()
    pltpu.sync_copy(acc_sh.at[pl.ds(s * _ROWS_PT, _ROWS_PT)],
                    out_hbm.at[c, pl.ds(s * _ROWS_PT, _ROWS_PT)])

  return pl.kernel(
      body,
      out_type=jax.ShapeDtypeStruct((_NC, _NPAD, fh), jnp.float32),
      mesh=_sc_mesh(),
      scratch_types=[
          pltpu.VMEM((2, nbuf, _CHUNK), jnp.int32),
          pltpu.VMEM((2, nbuf, _CHUNK), jnp.int32),
          pltpu.VMEM((nbuf, _CHUNK, fh), jnp.float32),
          pltpu.VMEM_SHARED((_NPAD, fh), jnp.float32),
          pltpu.SemaphoreType.DMA((2, 2)),
          pltpu.SemaphoreType.DMA((nbuf,)),
          pltpu.SemaphoreType.DMA((nbuf,)),
      ],
      compiler_params=_sc_params(),
  )


def _xw_call(x_pad, w1):
  nb = _NPAD // _BR
  fo = w1.shape[1]

  def body(x_ref, w_ref, u_ref):
    u_ref[...] = jnp.dot(x_ref[...], w_ref[...],
                         preferred_element_type=jnp.float32)

  return pl.pallas_call(
      body,
      grid=(nb,),
      in_specs=[
          pl.BlockSpec((_BR, 128), lambda i: (i, 0)),
          pl.BlockSpec((128, fo), lambda i: (0, 0)),
      ],
      out_specs=pl.BlockSpec((_BR, fo), lambda i: (i, 0)),
      out_shape=jax.ShapeDtypeStruct((_NPAD, fo), jnp.float32),
  )(x_pad, w1)


def _prep_call(deg_parts, u):
  nb = _NPAD // _BR
  fo = u.shape[1]

  def body(deg_ref, u_ref, dinv_ref, g_ref):
    i = pl.program_id(0)
    d = deg_ref[0] + deg_ref[1] + 1.0
    row = lax.broadcasted_iota(jnp.int32, (_BR, 1), 0) + i * _BR
    dinv = jnp.where(row < _N, lax.rsqrt(d), 0.0)
    dinv_ref[...] = jnp.broadcast_to(dinv, (_BR, 128))
    g_ref[...] = dinv * u_ref[...]

  return pl.pallas_call(
      body,
      grid=(nb,),
      in_specs=[
          pl.BlockSpec((_NC, _BR, 1), lambda i: (0, i, 0)),
          pl.BlockSpec((_BR, fo), lambda i: (i, 0)),
      ],
      out_specs=[
          pl.BlockSpec((_BR, 128), lambda i: (i, 0)),
          pl.BlockSpec((_BR, fo), lambda i: (i, 0)),
      ],
      out_shape=[
          jax.ShapeDtypeStruct((_NPAD, 128), jnp.float32),
          jax.ShapeDtypeStruct((_NPAD, fo), jnp.float32),
      ],
  )(deg_parts, u)


def _mid_call(parts, g, dinv, b2d, w):
  fh = g.shape[2]
  fo2 = w.shape[1]
  fh2 = fo2 // 2
  nb = _NPAD // _BR

  def body(p_ref, g_ref, dinv_ref, b_ref, w_ref, o_ref):
    agg = jnp.concatenate([p_ref[0] + g_ref[0], p_ref[1] + g_ref[1]], axis=1)
    dv = dinv_ref[...]
    h = jnp.maximum(dv[:, :2 * fh] * agg + b_ref[...], 0.0)
    gn = dv[:, :fo2] * jnp.dot(h, w_ref[...],
                               preferred_element_type=jnp.float32)
    o_ref[0] = gn[:, :fh2]
    o_ref[1] = gn[:, fh2:]

  return pl.pallas_call(
      body,
      grid=(nb,),
      in_specs=[
          pl.BlockSpec((_NC, _BR, fh), lambda i: (0, i, 0)),
          pl.BlockSpec((_NC, _BR, fh), lambda i: (0, i, 0)),
          pl.BlockSpec((_BR, 128), lambda i: (i, 0)),
          pl.BlockSpec((1, 2 * fh), lambda i: (0, 0)),
          pl.BlockSpec((2 * fh, fo2), lambda i: (0, 0)),
      ],
      out_specs=pl.BlockSpec((_NC, _BR, fh2), lambda i: (0, i, 0)),
      out_shape=jax.ShapeDtypeStruct((_NC, _NPAD, fh2), jnp.float32),
  )(parts, g, dinv, b2d, w)


def _mid_es_call(parts, g, dinv, b2d, w, split_out):
  fo = g.shape[1]
  fo2 = w.shape[1]
  fh2 = fo2 // 2
  nb = _NPAD // _BR

  def body(p_ref, g_ref, dinv_ref, b_ref, w_ref, o_ref):
    agg = p_ref[0] + p_ref[1] + g_ref[...]
    dv = dinv_ref[...]
    h = jnp.maximum(dv[:, :fo] * agg + b_ref[...], 0.0)
    gn = dv[:, :fo2] * jnp.dot(h, w_ref[...],
                               preferred_element_type=jnp.float32)
    if split_out:
      o_ref[0] = gn[:, :fh2]
      o_ref[1] = gn[:, fh2:]
    else:
      o_ref[...] = gn

  if split_out:
    out_spec = pl.BlockSpec((_NC, _BR, fh2), lambda i: (0, i, 0))
    out_shape = jax.ShapeDtypeStruct((_NC, _NPAD, fh2), jnp.float32)
  else:
    out_spec = pl.BlockSpec((_BR, fo2), lambda i: (i, 0))
    out_shape = jax.ShapeDtypeStruct((_NPAD, fo2), jnp.float32)

  return pl.pallas_call(
      body,
      grid=(nb,),
      in_specs=[
          pl.BlockSpec((_NC, _BR, fo), lambda i: (0, i, 0)),
          pl.BlockSpec((_BR, fo), lambda i: (i, 0)),
          pl.BlockSpec((_BR, 128), lambda i: (i, 0)),
          pl.BlockSpec((1, fo), lambda i: (0, 0)),
          pl.BlockSpec((fo, fo2), lambda i: (0, 0)),
      ],
      out_specs=out_spec,
      out_shape=out_shape,
  )(parts, g, dinv, b2d, w)


def _final_call(parts, g, dinv, b2d, bat2d, lw1, lb1, lw2, lb2):
  nb = _NPAD // _BR
  fh = g.shape[2]

  def body(p_ref, g_ref, dinv_ref, b_ref, bat_ref, lw1_ref, lb1_ref, lw2_ref,
           lb2_ref, o_ref, acc_ref):
    i = pl.program_id(0)

    @pl.when(i == 0)
    def _init():
      acc_ref[...] = jnp.full((_NG, 128), -jnp.inf, jnp.float32)

    agg = jnp.concatenate([p_ref[0] + g_ref[0], p_ref[1] + g_ref[1]], axis=1)
    h = jnp.maximum(dinv_ref[...] * agg + b_ref[...], 0.0)
    row = lax.broadcasted_iota(jnp.int32, (_BR, 1), 0) + i * _BR
    valid = row < _N
    bat = bat_ref[...]
    bmin = jnp.min(jnp.where(valid, bat, _NG - 1))
    bmax = jnp.max(jnp.where(valid, bat, 0))
    gcol = lax.broadcasted_iota(jnp.int32, (_NG, 1), 0)

    def gbody(gg, carry):
      m = (bat == gg) & valid
      red = jnp.max(jnp.where(m, h, -jnp.inf), axis=0, keepdims=True)
      acc_ref[...] = jnp.maximum(acc_ref[...],
                                 jnp.where(gcol == gg, red, -jnp.inf))
      return carry

    lax.fori_loop(bmin, bmax + 1, gbody, 0)

    @pl.when(i == nb - 1)
    def _fin():
      z = jnp.maximum(
          jnp.dot(acc_ref[...], lw1_ref[...],
                  preferred_element_type=jnp.float32) + lb1_ref[...], 0.0)
      o_ref[...] = jnp.dot(z, lw2_ref[...],
                           preferred_element_type=jnp.float32) + lb2_ref[...]

  return pl.pallas_call(
      body,
      grid=(nb,),
      in_specs=[
          pl.BlockSpec((_NC, _BR, fh), lambda i: (0, i, 0)),
          pl.BlockSpec((_NC, _BR, fh), lambda i: (0, i, 0)),
          pl.BlockSpec((_BR, 128), lambda i: (i, 0)),
          pl.BlockSpec((1, 128), lambda i: (0, 0)),
          pl.BlockSpec((_BR, 1), lambda i: (i, 0)),
          pl.BlockSpec((128, 64), lambda i: (0, 0)),
          pl.BlockSpec((1, 64), lambda i: (0, 0)),
          pl.BlockSpec((64, 10), lambda i: (0, 0)),
          pl.BlockSpec((1, 10), lambda i: (0, 0)),
      ],
      out_specs=pl.BlockSpec((_NG, 10), lambda i: (0, 0)),
      out_shape=jax.ShapeDtypeStruct((_NG, 10), jnp.float32),
      scratch_shapes=[pltpu.VMEM((_NG, 128), jnp.float32)],
  )(parts, g, dinv, b2d, bat2d, lw1, lb1, lw2, lb2)


def _pad_idx(n):
  return _N + (jnp.arange(n, dtype=jnp.int32) % (_NPAD - _N))


def kernel(x, edge_index, batch, W1, b1, W2, b2, W3, b3, W4, b4, W5, b5,
           W6, b6, lw1, lb1, lw2, lb2):
  e = edge_index.shape[1]
  src = edge_index[0].astype(jnp.int32)
  dst = edge_index[1].astype(jnp.int32)

  r_es = -(-e // (_NW * _CHUNK))
  r_es = -(-r_es // (2 * _NBUF)) * (2 * _NBUF)
  pad_es = _pad_idx(_NW * r_es * _CHUNK - e)
  src_es = jnp.concatenate([src, pad_es]).reshape(_NC, _NS, r_es, _CHUNK)
  dst_es = jnp.concatenate([dst, pad_es]).reshape(_NC, _NS, r_es, _CHUNK)
  dst_dg = dst_es.reshape(_NW, r_es, _CHUNK)

  r_ag = -(-e // (_NS * _CHUNK))
  r_ag = -(-r_ag // (2 * _NBUF)) * (2 * _NBUF)
  pad_ag = _pad_idx(_NS * r_ag * _CHUNK - e)
  src_t = jnp.concatenate([src, pad_ag]).reshape(_NS, r_ag, _CHUNK)
  dst_t = jnp.concatenate([dst, pad_ag]).reshape(_NS, r_ag, _CHUNK)
  srcp = jnp.stack([src_t, src_t + _NPAD])
  dstp = jnp.stack([dst_t, dst_t])

  x_pad = jnp.pad(x, ((0, _NPAD - _N), (0, 0)))
  bat2d = jnp.pad(batch.astype(jnp.int32), (0, _NPAD - _N),
                  constant_values=_NG - 1).reshape(_NPAD, 1)
  ones_c = jnp.ones((_CHUNK,), jnp.float32)
  zeros_r = jnp.zeros((_ROWS_PT,), jnp.float32)

  u1 = _xw_call(x_pad, W1)
  deg_parts = _deg_kernel(r_es)(dst_dg, ones_c, zeros_r)
  dinv, g = _prep_call(deg_parts.reshape(_NC, _NPAD, 1), u1)

  parts = _agg_kernel(16, r_es)(g, src_es, dst_es,
                                jnp.zeros((_ROWS_PT, 16), jnp.float32))
  g = _mid_es_call(parts, g, dinv, b1.reshape(1, -1), W2, False)
  parts = _agg_kernel(32, r_es)(g, src_es, dst_es,
                                jnp.zeros((_ROWS_PT, 32), jnp.float32))
  g = _mid_es_call(parts, g, dinv, b2.reshape(1, -1), W3, False)
  parts = _agg_kernel(48, r_es)(g, src_es, dst_es,
                                jnp.zeros((_ROWS_PT, 48), jnp.float32))
  g = _mid_es_call(parts, g, dinv, b3.reshape(1, -1), W4, True)

  ws = [W5, W6]
  bs = [b4, b5]
  for l in range(3):
    fh = g.shape[2]
    zeros_z = jnp.zeros((_ROWS_PT, fh), jnp.float32)
    parts = _agg_kernel(fh, r_ag)(g.reshape(_NC * _NPAD, fh), srcp, dstp,
                                  zeros_z)
    if l < 2:
      g = _mid_call(parts, g, dinv, bs[l].reshape(1, -1), ws[l])

  return _final_call(parts, g, dinv, b6.reshape(1, -1), bat2d,
                     lw1, lb1.reshape(1, -1), lw2, lb2.reshape(1, -1))


# speedup vs baseline: 40.7406x; 1.0096x over previous
import functools

import jax
import jax.numpy as jnp
from jax import lax
from jax.experimental import pallas as pl
from jax.experimental.pallas import tpu as pltpu
from jax.experimental.pallas import tpu_sc as plsc

_N = 10000
_NPAD = 10240
_NG = 64
_NC = 2
_NS = 16
_NW = _NC * _NS
_CHUNK = 128
_NBUF = 8
_ROWS_PT = _NPAD // _NS
_BR = 2048


def _sc_params():
  return pltpu.CompilerParams(use_tc_tiling_on_sc=False)


def _sc_mesh():
  return plsc.VectorSubcoreMesh(core_axis_name="c", subcore_axis_name="s")


@functools.lru_cache(maxsize=None)
def _deg_kernel(rounds: int):
  def body(dst_hbm, ones_hbm, zeros_hbm, out_hbm, dst_v, ones_v, acc_sh, sem):
    c = lax.axis_index("c")
    s = lax.axis_index("s")
    wid = c * _NS + s
    pltpu.sync_copy(dst_hbm.at[wid], dst_v)
    pltpu.sync_copy(ones_hbm, ones_v)
    pltpu.sync_copy(zeros_hbm, acc_sh.at[pl.ds(s * _ROWS_PT, _ROWS_PT)])
    plsc.subcore_barrier()

    def scat(j, carry):
      pltpu.async_copy(ones_v, acc_sh.at[dst_v.at[j]], sem, add=True)
      return carry

    lax.fori_loop(0, rounds, scat, 0)

    def drain(j, carry):
      pltpu.make_async_copy(ones_v, acc_sh.at[dst_v.at[j]], sem).wait()
      return carry

    lax.fori_loop(0, rounds, drain, 0)
    plsc.subcore_barrier()
    pltpu.sync_copy(acc_sh.at[pl.ds(s * _ROWS_PT, _ROWS_PT)],
                    out_hbm.at[c, pl.ds(s * _ROWS_PT, _ROWS_PT)])

  return pl.kernel(
      body,
      out_type=jax.ShapeDtypeStruct((_NC, _NPAD), jnp.float32),
      mesh=_sc_mesh(),
      scratch_types=[
          pltpu.VMEM((rounds, _CHUNK), jnp.int32),
          pltpu.VMEM((_CHUNK,), jnp.float32),
          pltpu.VMEM_SHARED((_NPAD,), jnp.float32),
          pltpu.SemaphoreType.DMA,
      ],
      compiler_params=_sc_params(),
  )


@functools.lru_cache(maxsize=None)
def _agg_kernel(fh: int, rounds: int):
  nbuf = _NBUF
  ngroups = rounds // nbuf
  assert ngroups % 2 == 0

  def body(g_hbm, src_hbm, dst_hbm, zeros_hbm, out_hbm,
           sidx, didx, buf_v, acc_sh, isem, gsem, ssem):
    c = lax.axis_index("c")
    s = lax.axis_index("s")

    pltpu.sync_copy(zeros_hbm, acc_sh.at[pl.ds(s * _ROWS_PT, _ROWS_PT)])
    pltpu.async_copy(src_hbm.at[c, s, pl.ds(0, nbuf)], sidx.at[0],
                     isem.at[0, 0])
    pltpu.async_copy(dst_hbm.at[c, s, pl.ds(0, nbuf)], didx.at[0],
                     isem.at[0, 1])
    plsc.subcore_barrier()

    def pair(hj, carry):
      for par in range(2):
        gi = 2 * hj + par
        base = gi * nbuf

        pltpu.make_async_copy(src_hbm.at[c, s, pl.ds(base, nbuf)],
                              sidx.at[par], isem.at[par, 0]).wait()
        pltpu.make_async_copy(dst_hbm.at[c, s, pl.ds(base, nbuf)],
                              didx.at[par], isem.at[par, 1]).wait()

        for b in range(nbuf):
          @pl.when(gi > 0)
          def _drain_prev():
            pltpu.make_async_copy(buf_v.at[b], acc_sh.at[didx.at[1 - par, b]],
                                  ssem.at[b]).wait()

          pltpu.async_copy(g_hbm.at[sidx.at[par, b]], buf_v.at[b],
                           gsem.at[b])

        @pl.when(gi + 1 < ngroups)
        def _prefetch():
          nb = (gi + 1) * nbuf
          pltpu.async_copy(src_hbm.at[c, s, pl.ds(nb, nbuf)],
                           sidx.at[1 - par], isem.at[1 - par, 0])
          pltpu.async_copy(dst_hbm.at[c, s, pl.ds(nb, nbuf)],
                           didx.at[1 - par], isem.at[1 - par, 1])

        for b in range(nbuf):
          pltpu.make_async_copy(g_hbm.at[sidx.at[par, b]], buf_v.at[b],
                                gsem.at[b]).wait()
          pltpu.async_copy(buf_v.at[b], acc_sh.at[didx.at[par, b]],
                           ssem.at[b], add=True)
      return carry

    lax.fori_loop(0, ngroups // 2, pair, 0)
    for b in range(nbuf):
      pltpu.make_async_copy(buf_v.at[b], acc_sh.at[didx.at[1, b]],
                            ssem.at[b]).wait()
    plsc.subcore_barrier()
    pltpu.sync_copy(acc_sh.at[pl.ds(s * _ROWS_PT, _ROWS_PT)],
                    out_hbm.at[c, pl.ds(s * _ROWS_PT, _ROWS_PT)])

  return pl.kernel(
      body,
      out_type=jax.ShapeDtypeStruct((_NC, _NPAD, fh), jnp.float32),
      mesh=_sc_mesh(),
      scratch_types=[
          pltpu.VMEM((2, nbuf, _CHUNK), jnp.int32),
          pltpu.VMEM((2, nbuf, _CHUNK), jnp.int32),
          pltpu.VMEM((nbuf, _CHUNK, fh), jnp.float32),
          pltpu.VMEM_SHARED((_NPAD, fh), jnp.float32),
          pltpu.SemaphoreType.DMA((2, 2)),
          pltpu.SemaphoreType.DMA((nbuf,)),
          pltpu.SemaphoreType.DMA((nbuf,)),
      ],
      compiler_params=_sc_params(),
  )


def _xw_call(x_pad, w1):
  nb = _NPAD // _BR
  fo = w1.shape[1]

  def body(x_ref, w_ref, u_ref):
    u_ref[...] = jnp.dot(x_ref[...], w_ref[...],
                         preferred_element_type=jnp.float32)

  return pl.pallas_call(
      body,
      grid=(nb,),
      in_specs=[
          pl.BlockSpec((_BR, 128), lambda i: (i, 0)),
          pl.BlockSpec((128, fo), lambda i: (0, 0)),
      ],
      out_specs=pl.BlockSpec((_BR, fo), lambda i: (i, 0)),
      out_shape=jax.ShapeDtypeStruct((_NPAD, fo), jnp.float32),
  )(x_pad, w1)


def _prep_call(deg_parts, u):
  nb = _NPAD // _BR
  fo = u.shape[1]

  def body(deg_ref, u_ref, dinv_ref, g_ref):
    i = pl.program_id(0)
    d = deg_ref[0] + deg_ref[1] + 1.0
    row = lax.broadcasted_iota(jnp.int32, (_BR, 1), 0) + i * _BR
    dinv = jnp.where(row < _N, lax.rsqrt(d), 0.0)
    dinv_ref[...] = jnp.broadcast_to(dinv, (_BR, 128))
    g_ref[...] = dinv * u_ref[...]

  return pl.pallas_call(
      body,
      grid=(nb,),
      in_specs=[
          pl.BlockSpec((_NC, _BR, 1), lambda i: (0, i, 0)),
          pl.BlockSpec((_BR, fo), lambda i: (i, 0)),
      ],
      out_specs=[
          pl.BlockSpec((_BR, 128), lambda i: (i, 0)),
          pl.BlockSpec((_BR, fo), lambda i: (i, 0)),
      ],
      out_shape=[
          jax.ShapeDtypeStruct((_NPAD, 128), jnp.float32),
          jax.ShapeDtypeStruct((_NPAD, fo), jnp.float32),
      ],
  )(deg_parts, u)


def _mid_call(parts, g, dinv, b2d, w):
  fh = g.shape[2]
  fo2 = w.shape[1]
  fh2 = fo2 // 2
  nb = _NPAD // _BR

  def body(p_ref, g_ref, dinv_ref, b_ref, w_ref, o_ref):
    agg = jnp.concatenate([p_ref[0] + g_ref[0], p_ref[1] + g_ref[1]], axis=1)
    dv = dinv_ref[...]
    h = jnp.maximum(dv[:, :2 * fh] * agg + b_ref[...], 0.0)
    gn = dv[:, :fo2] * jnp.dot(h, w_ref[...],
                               preferred_element_type=jnp.float32)
    o_ref[0] = gn[:, :fh2]
    o_ref[1] = gn[:, fh2:]

  return pl.pallas_call(
      body,
      grid=(nb,),
      in_specs=[
          pl.BlockSpec((_NC, _BR, fh), lambda i: (0, i, 0)),
          pl.BlockSpec((_NC, _BR, fh), lambda i: (0, i, 0)),
          pl.BlockSpec((_BR, 128), lambda i: (i, 0)),
          pl.BlockSpec((1, 2 * fh), lambda i: (0, 0)),
          pl.BlockSpec((2 * fh, fo2), lambda i: (0, 0)),
      ],
      out_specs=pl.BlockSpec((_NC, _BR, fh2), lambda i: (0, i, 0)),
      out_shape=jax.ShapeDtypeStruct((_NC, _NPAD, fh2), jnp.float32),
  )(parts, g, dinv, b2d, w)


def _mid_es_call(parts, g, dinv, b2d, w, split_out):
  fo = g.shape[1]
  fo2 = w.shape[1]
  fh2 = fo2 // 2
  nb = _NPAD // _BR

  def body(p_ref, g_ref, dinv_ref, b_ref, w_ref, o_ref):
    agg = p_ref[0] + p_ref[1] + g_ref[...]
    dv = dinv_ref[...]
    h = jnp.maximum(dv[:, :fo] * agg + b_ref[...], 0.0)
    gn = dv[:, :fo2] * jnp.dot(h, w_ref[...],
                               preferred_element_type=jnp.float32)
    if split_out:
      o_ref[0] = gn[:, :fh2]
      o_ref[1] = gn[:, fh2:]
    else:
      o_ref[...] = gn

  if split_out:
    out_spec = pl.BlockSpec((_NC, _BR, fh2), lambda i: (0, i, 0))
    out_shape = jax.ShapeDtypeStruct((_NC, _NPAD, fh2), jnp.float32)
  else:
    out_spec = pl.BlockSpec((_BR, fo2), lambda i: (i, 0))
    out_shape = jax.ShapeDtypeStruct((_NPAD, fo2), jnp.float32)

  return pl.pallas_call(
      body,
      grid=(nb,),
      in_specs=[
          pl.BlockSpec((_NC, _BR, fo), lambda i: (0, i, 0)),
          pl.BlockSpec((_BR, fo), lambda i: (i, 0)),
          pl.BlockSpec((_BR, 128), lambda i: (i, 0)),
          pl.BlockSpec((1, fo), lambda i: (0, 0)),
          pl.BlockSpec((fo, fo2), lambda i: (0, 0)),
      ],
      out_specs=out_spec,
      out_shape=out_shape,
  )(parts, g, dinv, b2d, w)


def _final_call(parts, g, dinv, b2d, bat2d, lw1, lb1, lw2, lb2):
  nb = _NPAD // _BR
  fh = g.shape[2]

  def body(p_ref, g_ref, dinv_ref, b_ref, bat_ref, lw1_ref, lb1_ref, lw2_ref,
           lb2_ref, o_ref, acc_ref):
    i = pl.program_id(0)

    @pl.when(i == 0)
    def _init():
      acc_ref[...] = jnp.full((_NG, 128), -jnp.inf, jnp.float32)

    agg = jnp.concatenate([p_ref[0] + g_ref[0], p_ref[1] + g_ref[1]], axis=1)
    h = jnp.maximum(dinv_ref[...] * agg + b_ref[...], 0.0)
    row = lax.broadcasted_iota(jnp.int32, (_BR, 1), 0) + i * _BR
    valid = row < _N
    bat = bat_ref[...]
    bmin = jnp.min(jnp.where(valid, bat, _NG - 1))
    bmax = jnp.max(jnp.where(valid, bat, 0))
    gcol = lax.broadcasted_iota(jnp.int32, (_NG, 1), 0)

    def gbody(gg, carry):
      m = (bat == gg) & valid
      red = jnp.max(jnp.where(m, h, -jnp.inf), axis=0, keepdims=True)
      acc_ref[...] = jnp.maximum(acc_ref[...],
                                 jnp.where(gcol == gg, red, -jnp.inf))
      return carry

    lax.fori_loop(bmin, bmax + 1, gbody, 0)

    @pl.when(i == nb - 1)
    def _fin():
      z = jnp.maximum(
          jnp.dot(acc_ref[...], lw1_ref[...],
                  preferred_element_type=jnp.float32) + lb1_ref[...], 0.0)
      o_ref[...] = jnp.dot(z, lw2_ref[...],
                           preferred_element_type=jnp.float32) + lb2_ref[...]

  return pl.pallas_call(
      body,
      grid=(nb,),
      in_specs=[
          pl.BlockSpec((_NC, _BR, fh), lambda i: (0, i, 0)),
          pl.BlockSpec((_NC, _BR, fh), lambda i: (0, i, 0)),
          pl.BlockSpec((_BR, 128), lambda i: (i, 0)),
          pl.BlockSpec((1, 128), lambda i: (0, 0)),
          pl.BlockSpec((_BR, 1), lambda i: (i, 0)),
          pl.BlockSpec((128, 64), lambda i: (0, 0)),
          pl.BlockSpec((1, 64), lambda i: (0, 0)),
          pl.BlockSpec((64, 10), lambda i: (0, 0)),
          pl.BlockSpec((1, 10), lambda i: (0, 0)),
      ],
      out_specs=pl.BlockSpec((_NG, 10), lambda i: (0, 0)),
      out_shape=jax.ShapeDtypeStruct((_NG, 10), jnp.float32),
      scratch_shapes=[pltpu.VMEM((_NG, 128), jnp.float32)],
  )(parts, g, dinv, b2d, bat2d, lw1, lb1, lw2, lb2)


def _pad_idx(n):
  return _N + (jnp.arange(n, dtype=jnp.int32) % (_NPAD - _N))


def kernel(x, edge_index, batch, W1, b1, W2, b2, W3, b3, W4, b4, W5, b5,
           W6, b6, lw1, lb1, lw2, lb2):
  e = edge_index.shape[1]
  src = edge_index[0].astype(jnp.int32)
  dst = edge_index[1].astype(jnp.int32)

  r_es = -(-e // (_NW * _CHUNK))
  r_es = -(-r_es // (2 * _NBUF)) * (2 * _NBUF)
  pad_es = _pad_idx(_NW * r_es * _CHUNK - e)
  src_es = jnp.concatenate([src, pad_es]).reshape(_NC, _NS, r_es, _CHUNK)
  dst_es = jnp.concatenate([dst, pad_es]).reshape(_NC, _NS, r_es, _CHUNK)
  dst_dg = dst_es.reshape(_NW, r_es, _CHUNK)

  r_ag = -(-e // (_NS * _CHUNK))
  r_ag = -(-r_ag // (2 * _NBUF)) * (2 * _NBUF)
  pad_ag = _pad_idx(_NS * r_ag * _CHUNK - e)
  src_t = jnp.concatenate([src, pad_ag]).reshape(_NS, r_ag, _CHUNK)
  dst_t = jnp.concatenate([dst, pad_ag]).reshape(_NS, r_ag, _CHUNK)
  srcp = jnp.stack([src_t, src_t + _NPAD])
  dstp = jnp.stack([dst_t, dst_t])

  x_pad = jnp.pad(x, ((0, _NPAD - _N), (0, 0)))
  bat2d = jnp.pad(batch.astype(jnp.int32), (0, _NPAD - _N),
                  constant_values=_NG - 1).reshape(_NPAD, 1)
  ones_c = jnp.ones((_CHUNK,), jnp.float32)
  zeros_r = jnp.zeros((_ROWS_PT,), jnp.float32)

  u1 = _xw_call(x_pad, W1)
  deg_parts = _deg_kernel(r_es)(dst_dg, ones_c, zeros_r)
  dinv, g = _prep_call(deg_parts.reshape(_NC, _NPAD, 1), u1)

  parts = _agg_kernel(16, r_es)(g, src_es, dst_es,
                                jnp.zeros((_ROWS_PT, 16), jnp.float32))
  g = _mid_es_call(parts, g, dinv, b1.reshape(1, -1), W2, False)
  parts = _agg_kernel(32, r_es)(g, src_es, dst_es,
                                jnp.zeros((_ROWS_PT, 32), jnp.float32))
  g = _mid_es_call(parts, g, dinv, b2.reshape(1, -1), W3, False)
  parts = _agg_kernel(48, r_es)(g, src_es, dst_es,
                                jnp.zeros((_ROWS_PT, 48), jnp.float32))
  g = _mid_es_call(parts, g, dinv, b3.reshape(1, -1), W4, False)
  parts = _agg_kernel(64, r_es)(g, src_es, dst_es,
                                jnp.zeros((_ROWS_PT, 64), jnp.float32))
  g = _mid_es_call(parts, g, dinv, b4.reshape(1, -1), W5, True)

  ws = [W6]
  bs = [b5]
  for l in range(2):
    fh = g.shape[2]
    zeros_z = jnp.zeros((_ROWS_PT, fh), jnp.float32)
    parts = _agg_kernel(fh, r_ag)(g.reshape(_NC * _NPAD, fh), srcp, dstp,
                                  zeros_z)
    if l < 1:
      g = _mid_call(parts, g, dinv, bs[l].reshape(1, -1), ws[l])

  return _final_call(parts, g, dinv, b6.reshape(1, -1), bat2d,
                     lw1, lb1.reshape(1, -1), lw2, lb2.reshape(1, -1))
